# baseline (speedup 1.0000x reference)
import functools

import jax
import jax.numpy as jnp
from jax import lax
from jax.experimental import pallas as pl
from jax.experimental.pallas import tpu as pltpu
from jax.experimental.pallas import tpu_sc as plsc

N = 10000
D = 128
NC = 2
NS = 16
CH = 128
PADR = 128
GRP = 8


def _round_up(a, b):
    return (a + b - 1) // b * b


_ROWS = 2000


def _dense_body(h_ref, wl_ref, wr_ref, y_ref, r_ref):
    h = h_ref[...]
    y_ref[...] = jnp.dot(h, wl_ref[...], preferred_element_type=jnp.float32)
    r_ref[...] = jnp.dot(h, wr_ref[...], preferred_element_type=jnp.float32)


def _tc_dense(h, Wl, Wr):
    n = h.shape[0]
    return pl.pallas_call(
        _dense_body,
        grid=(n // _ROWS,),
        in_specs=[
            pl.BlockSpec((_ROWS, D), lambda i: (i, 0)),
            pl.BlockSpec((D, D), lambda i: (0, 0)),
            pl.BlockSpec((D, D), lambda i: (0, 0)),
        ],
        out_specs=[pl.BlockSpec((_ROWS, D), lambda i: (i, 0))] * 2,
        out_shape=[jax.ShapeDtypeStruct((n, D), jnp.float32)] * 2,
    )(h, Wl, Wr)


def _combine_body(p_ref, c_ref, rp_ref, b_ref, wl_ref, wr_ref, y_ref, r_ref):
    cnt = jnp.maximum(c_ref[0][:, 0:1] + c_ref[1][:, 0:1], 1.0)
    mean = (p_ref[0] + p_ref[1]) / cnt
    h = jnp.maximum(mean + rp_ref[...] + b_ref[...], 0.0)
    y_ref[...] = jnp.dot(h, wl_ref[...], preferred_element_type=jnp.float32)
    r_ref[...] = jnp.dot(h, wr_ref[...], preferred_element_type=jnp.float32)


def _tc_combine(P, C, r_prev, b, Wl, Wr):
    n = r_prev.shape[0]
    return pl.pallas_call(
        _combine_body,
        grid=(n // _ROWS,),
        in_specs=[
            pl.BlockSpec((2, _ROWS, D), lambda i: (0, i, 0)),
            pl.BlockSpec((2, _ROWS, D), lambda i: (0, i, 0)),
            pl.BlockSpec((_ROWS, D), lambda i: (i, 0)),
            pl.BlockSpec((1, D), lambda i: (0, 0)),
            pl.BlockSpec((D, D), lambda i: (0, 0)),
            pl.BlockSpec((D, D), lambda i: (0, 0)),
        ],
        out_specs=[pl.BlockSpec((_ROWS, D), lambda i: (i, 0))] * 2,
        out_shape=[jax.ShapeDtypeStruct((n, D), jnp.float32)] * 2,
    )(P, C, r_prev, b, Wl, Wr)


def _final_body(p_ref, c_ref, rp_ref, b_ref, o_ref):
    cnt = jnp.maximum(c_ref[0][:, 0:1] + c_ref[1][:, 0:1], 1.0)
    o_ref[...] = (p_ref[0] + p_ref[1]) / cnt + rp_ref[...] + b_ref[...]


def _tc_final(P, C, r_prev, b):
    n = r_prev.shape[0]
    return pl.pallas_call(
        _final_body,
        grid=(n // _ROWS,),
        in_specs=[
            pl.BlockSpec((2, _ROWS, D), lambda i: (0, i, 0)),
            pl.BlockSpec((2, _ROWS, D), lambda i: (0, i, 0)),
            pl.BlockSpec((_ROWS, D), lambda i: (i, 0)),
            pl.BlockSpec((1, D), lambda i: (0, 0)),
        ],
        out_specs=pl.BlockSpec((_ROWS, D), lambda i: (i, 0)),
        out_shape=jax.ShapeDtypeStruct((n, D), jnp.float32),
    )(P, C, r_prev, b)


def _zcomb_body(p_ref, c_ref, rp_ref, b_ref, z_ref, h_ref):
    cnt = jnp.maximum(c_ref[0][:, 0:1] + c_ref[1][:, 0:1], 1.0)
    z = (p_ref[0] + p_ref[1]) / cnt + rp_ref[...] + b_ref[...]
    z_ref[...] = z
    h_ref[...] = jnp.maximum(z, 0.0)


def _tc_zcombine(P, C, r_prev, b):
    n = r_prev.shape[0]
    return pl.pallas_call(
        _zcomb_body,
        grid=(n // _ROWS,),
        in_specs=[
            pl.BlockSpec((2, _ROWS, D), lambda i: (0, i, 0)),
            pl.BlockSpec((2, _ROWS, D), lambda i: (0, i, 0)),
            pl.BlockSpec((_ROWS, D), lambda i: (i, 0)),
            pl.BlockSpec((1, D), lambda i: (0, 0)),
        ],
        out_specs=[pl.BlockSpec((_ROWS, D), lambda i: (i, 0))] * 2,
        out_shape=[jax.ShapeDtypeStruct((n, D), jnp.float32)] * 2,
    )(P, C, r_prev, b)


def _make_sc_segsum(n_nodes, rows_per_tile, rpt0=None):
    n_acc = n_nodes + PADR
    base = (n_nodes // NS) // 8 * 8
    wlen = n_nodes - base * (NS - 1)
    assert wlen % CH == 0 and wlen % 8 == 0

    mesh = plsc.VectorSubcoreMesh(core_axis_name="c", subcore_axis_name="s")
    out_type = jax.ShapeDtypeStruct((NC, n_nodes, D), jnp.float32)

    scratch = [
        pltpu.VMEM((GRP, CH), jnp.int32),
        pltpu.VMEM((GRP, CH), jnp.int32),
        pltpu.VMEM((2, CH, D), jnp.float32),
        pltpu.VMEM_SHARED((n_acc, D), jnp.float32),
        pltpu.SemaphoreType.DMA,
    ]

    if rpt0 is None:
        rpt0 = rows_per_tile
    rpt1 = 2 * rows_per_tile - rpt0
    assert rpt0 % GRP == 0 and rpt1 % GRP == 0
    ng0, ng1 = rpt0 // GRP, rpt1 // GRP

    def body(y_hbm, src_hbm, dst_hbm, psum_hbm, src_v, dst_v, gbuf, acc, gsem):
        c = lax.axis_index("c")
        s = lax.axis_index("s")
        row0 = jnp.where(c == 0, s * rpt0, NS * rpt0 + s * rpt1)
        ng = jnp.where(c == 0, ng0, ng1)

        with jax.named_scope("zero_phase"):
            def zrow(i, _):
                for j in range(D // 16):
                    gbuf[0, i, pl.ds(j * 16, 16)] = jnp.zeros((16,),
                                                              jnp.float32)
                return 0

            lax.fori_loop(0, CH, zrow, 0)

            zb = s * base
            for j in range(wlen // CH):
                pltpu.sync_copy(gbuf.at[0], acc.at[pl.ds(zb + j * CH, CH)])
            @pl.when(s == 0)
            def _():
                pltpu.sync_copy(gbuf.at[0, pl.ds(0, PADR)],
                                acc.at[pl.ds(n_nodes, PADR)])

        with jax.named_scope("zero_barrier"):
            plsc.subcore_barrier()

        def group(g, _):
            @pl.when(g < ng)
            def _():
                gr = row0 + g * GRP
                pltpu.sync_copy(src_hbm.at[pl.ds(gr, GRP)], src_v)
                pltpu.sync_copy(dst_hbm.at[pl.ds(gr, GRP)], dst_v)
                pending = pltpu.async_copy(y_hbm.at[src_v.at[0]], gbuf.at[0],
                                           gsem)
                for j in range(GRP):
                    pending.wait()
                    if j < GRP - 1:
                        pending = pltpu.async_copy(
                            y_hbm.at[src_v.at[j + 1]], gbuf.at[(j + 1) % 2],
                            gsem)
                    pltpu.sync_copy(gbuf.at[j % 2], acc.at[dst_v.at[j]],
                                    add=True)
            return 0

        with jax.named_scope("edge_loop"):
            lax.fori_loop(0, max(ng0, ng1), group, 0)

        with jax.named_scope("end_barrier"):
            plsc.subcore_barrier()

        with jax.named_scope("writeback"):
            wb = s * base
            pltpu.sync_copy(acc.at[pl.ds(wb, wlen)],
                            psum_hbm.at[c, pl.ds(wb, wlen)])

    return pl.kernel(body, out_type=out_type, mesh=mesh, scratch_types=scratch)


def _make_sc_counts(n_nodes, rows_per_tile):
    n_acc = n_nodes + PADR
    base = (n_nodes // NS) // 8 * 8
    wlen = n_nodes - base * (NS - 1)

    mesh = plsc.VectorSubcoreMesh(core_axis_name="c", subcore_axis_name="s")
    out_type = jax.ShapeDtypeStruct((NC, n_nodes, D), jnp.float32)
    scratch = [
        pltpu.VMEM((GRP, CH), jnp.int32),
        pltpu.VMEM((CH, D), jnp.float32),
        pltpu.VMEM_SHARED((n_acc, D), jnp.float32),
    ]

    def body(dst_hbm, cnt_hbm, dst_v, buf, cacc):
        c = lax.axis_index("c")
        s = lax.axis_index("s")
        row0 = (c * NS + s) * rows_per_tile

        def fill(val):
            def frow(i, _):
                for j in range(D // 16):
                    buf[i, pl.ds(j * 16, 16)] = jnp.full((16,), val, jnp.float32)
                return 0
            lax.fori_loop(0, CH, frow, 0)

        fill(0.0)
        zb = s * base
        for j in range(wlen // CH):
            pltpu.sync_copy(buf, cacc.at[pl.ds(zb + j * CH, CH)])
        @pl.when(s == 0)
        def _():
            pltpu.sync_copy(buf.at[pl.ds(0, PADR)], cacc.at[pl.ds(n_nodes, PADR)])
        fill(1.0)

        plsc.subcore_barrier()

        def group(g, _):
            pltpu.sync_copy(dst_hbm.at[pl.ds(row0 + g * GRP, GRP)], dst_v)
            for j in range(GRP):
                pltpu.sync_copy(buf, cacc.at[dst_v.at[j]], add=True)
            return 0

        lax.fori_loop(0, rows_per_tile // GRP, group, 0)

        plsc.subcore_barrier()
        wb = s * base
        pltpu.sync_copy(cacc.at[pl.ds(wb, wlen)],
                        cnt_hbm.at[c, pl.ds(wb, wlen)])

    return pl.kernel(body, out_type=out_type, mesh=mesh, scratch_types=scratch)


def kernel(x, edge_index, Wl0, bl0, Wr0, Wl1, bl1, Wr1, Wl2, bl2, Wr2):
    n = x.shape[0]
    e = edge_index.shape[1]

    e_pad = _round_up(e, NC * NS * CH * 8)
    src = edge_index[0]
    dst = edge_index[1]
    if e_pad != e:
        pad = e_pad - e
        src = jnp.concatenate([src, jnp.zeros((pad,), jnp.int32)])
        dst = jnp.concatenate(
            [dst, n + (jnp.arange(pad, dtype=jnp.int32) % PADR)])
    src2d = src.reshape(e_pad // CH, CH)
    dst2d = dst.reshape(e_pad // CH, CH)
    rows_per_tile = (e_pad // CH) // (NC * NS)

    sc_seg = _make_sc_segsum(n, rows_per_tile)
    sc_cnt = _make_sc_counts(n, rows_per_tile)

    b0 = bl0.reshape(1, D)
    b1 = bl1.reshape(1, D)
    b2 = bl2.reshape(1, D)

    C = sc_cnt(dst2d)
    WL = jnp.stack([Wl0, Wl1, Wl2])
    WR = jnp.stack([Wr0, Wr1, Wr2])
    B3 = jnp.stack([b0, b1, b2])

    def step(h, wlb):
        Wl, Wr, b = wlb
        y, r = _tc_dense(h, Wl, Wr)
        P = sc_seg(y, src2d, dst2d)
        z, h2 = _tc_zcombine(P, C, r, b)
        return h2, z

    _, zs = lax.scan(step, x, (WL, WR, B3))
    return zs[2]


# speedup vs baseline: 2.6215x; 2.6215x over previous
import functools

import jax
import jax.numpy as jnp
from jax import lax
from jax.experimental import pallas as pl
from jax.experimental.pallas import tpu as pltpu
from jax.experimental.pallas import tpu_sc as plsc

N = 10000
D = 128
NC = 2
NS = 16
CH = 128
PADR = 128
GRP = 8


def _round_up(a, b):
    return (a + b - 1) // b * b


_ROWS = 2000


def _dense_body(h_ref, wl_ref, wr_ref, y_ref, r_ref):
    h = h_ref[...]
    y_ref[...] = jnp.dot(h, wl_ref[...], preferred_element_type=jnp.float32)
    r_ref[...] = jnp.dot(h, wr_ref[...], preferred_element_type=jnp.float32)


def _tc_dense(h, Wl, Wr):
    n = h.shape[0]
    return pl.pallas_call(
        _dense_body,
        grid=(n // _ROWS,),
        in_specs=[
            pl.BlockSpec((_ROWS, D), lambda i: (i, 0)),
            pl.BlockSpec((D, D), lambda i: (0, 0)),
            pl.BlockSpec((D, D), lambda i: (0, 0)),
        ],
        out_specs=[pl.BlockSpec((_ROWS, D), lambda i: (i, 0))] * 2,
        out_shape=[jax.ShapeDtypeStruct((n, D), jnp.float32)] * 2,
    )(h, Wl, Wr)


def _combine_body(p_ref, c_ref, rp_ref, b_ref, wl_ref, wr_ref, y_ref, r_ref):
    cnt = jnp.maximum(c_ref[0][:, 0:1] + c_ref[1][:, 0:1], 1.0)
    mean = (p_ref[0] + p_ref[1]) / cnt
    h = jnp.maximum(mean + rp_ref[...] + b_ref[...], 0.0)
    y_ref[...] = jnp.dot(h, wl_ref[...], preferred_element_type=jnp.float32)
    r_ref[...] = jnp.dot(h, wr_ref[...], preferred_element_type=jnp.float32)


def _tc_combine(P, C, r_prev, b, Wl, Wr):
    n = r_prev.shape[0]
    return pl.pallas_call(
        _combine_body,
        grid=(n // _ROWS,),
        in_specs=[
            pl.BlockSpec((2, _ROWS, D), lambda i: (0, i, 0)),
            pl.BlockSpec((2, _ROWS, D), lambda i: (0, i, 0)),
            pl.BlockSpec((_ROWS, D), lambda i: (i, 0)),
            pl.BlockSpec((1, D), lambda i: (0, 0)),
            pl.BlockSpec((D, D), lambda i: (0, 0)),
            pl.BlockSpec((D, D), lambda i: (0, 0)),
        ],
        out_specs=[pl.BlockSpec((_ROWS, D), lambda i: (i, 0))] * 2,
        out_shape=[jax.ShapeDtypeStruct((n, D), jnp.float32)] * 2,
    )(P, C, r_prev, b, Wl, Wr)


def _final_body(p_ref, c_ref, rp_ref, b_ref, o_ref):
    cnt = jnp.maximum(c_ref[0][:, 0:1] + c_ref[1][:, 0:1], 1.0)
    o_ref[...] = (p_ref[0] + p_ref[1]) / cnt + rp_ref[...] + b_ref[...]


def _tc_final(P, C, r_prev, b):
    n = r_prev.shape[0]
    return pl.pallas_call(
        _final_body,
        grid=(n // _ROWS,),
        in_specs=[
            pl.BlockSpec((2, _ROWS, D), lambda i: (0, i, 0)),
            pl.BlockSpec((2, _ROWS, D), lambda i: (0, i, 0)),
            pl.BlockSpec((_ROWS, D), lambda i: (i, 0)),
            pl.BlockSpec((1, D), lambda i: (0, 0)),
        ],
        out_specs=pl.BlockSpec((_ROWS, D), lambda i: (i, 0)),
        out_shape=jax.ShapeDtypeStruct((n, D), jnp.float32),
    )(P, C, r_prev, b)


def _zcomb_body(p_ref, c_ref, rp_ref, b_ref, z_ref, h_ref):
    cnt = jnp.maximum(c_ref[0][:, 0:1] + c_ref[1][:, 0:1], 1.0)
    z = (p_ref[0] + p_ref[1]) / cnt + rp_ref[...] + b_ref[...]
    z_ref[...] = z
    h_ref[...] = jnp.maximum(z, 0.0)


def _tc_zcombine(P, C, r_prev, b):
    n = r_prev.shape[0]
    return pl.pallas_call(
        _zcomb_body,
        grid=(n // _ROWS,),
        in_specs=[
            pl.BlockSpec((2, _ROWS, D), lambda i: (0, i, 0)),
            pl.BlockSpec((2, _ROWS, D), lambda i: (0, i, 0)),
            pl.BlockSpec((_ROWS, D), lambda i: (i, 0)),
            pl.BlockSpec((1, D), lambda i: (0, 0)),
        ],
        out_specs=[pl.BlockSpec((_ROWS, D), lambda i: (i, 0))] * 2,
        out_shape=[jax.ShapeDtypeStruct((n, D), jnp.float32)] * 2,
    )(P, C, r_prev, b)


def _make_sc_segsum(n_nodes, rows_per_tile, rpt0=None):
    n_acc = n_nodes + PADR
    base = (n_nodes // NS) // 8 * 8
    wlen = n_nodes - base * (NS - 1)
    assert wlen % CH == 0 and wlen % 8 == 0

    mesh = plsc.VectorSubcoreMesh(core_axis_name="c", subcore_axis_name="s")
    out_type = jax.ShapeDtypeStruct((NC, n_nodes, D), jnp.float32)

    scratch = [
        pltpu.VMEM((GRP, CH), jnp.int32),
        pltpu.VMEM((GRP, CH), jnp.int32),
        pltpu.VMEM((2, CH, D), jnp.float32),
        pltpu.VMEM_SHARED((n_acc, D), jnp.float32),
        pltpu.SemaphoreType.DMA,
    ]

    if rpt0 is None:
        rpt0 = rows_per_tile
    rpt1 = 2 * rows_per_tile - rpt0
    assert rpt0 % GRP == 0 and rpt1 % GRP == 0
    ng0, ng1 = rpt0 // GRP, rpt1 // GRP

    def body(y_hbm, src_hbm, dst_hbm, psum_hbm, src_v, dst_v, gbuf, acc, gsem):
        c = lax.axis_index("c")
        s = lax.axis_index("s")
        row0 = jnp.where(c == 0, s * rpt0, NS * rpt0 + s * rpt1)
        ng = jnp.where(c == 0, ng0, ng1)

        with jax.named_scope("zero_phase"):
            def zrow(i, _):
                for j in range(D // 16):
                    gbuf[0, i, pl.ds(j * 16, 16)] = jnp.zeros((16,),
                                                              jnp.float32)
                return 0

            lax.fori_loop(0, CH, zrow, 0)

            zb = s * base
            for j in range(wlen // CH):
                pltpu.sync_copy(gbuf.at[0], acc.at[pl.ds(zb + j * CH, CH)])
            @pl.when(s == 0)
            def _():
                pltpu.sync_copy(gbuf.at[0, pl.ds(0, PADR)],
                                acc.at[pl.ds(n_nodes, PADR)])

        with jax.named_scope("zero_barrier"):
            plsc.subcore_barrier()

        def group(g, _):
            @pl.when(g < ng)
            def _():
                gr = row0 + g * GRP
                pltpu.sync_copy(src_hbm.at[pl.ds(gr, GRP)], src_v)
                pltpu.sync_copy(dst_hbm.at[pl.ds(gr, GRP)], dst_v)
                pending = pltpu.async_copy(y_hbm.at[src_v.at[0]], gbuf.at[0],
                                           gsem)
                for j in range(GRP):
                    pending.wait()
                    if j < GRP - 1:
                        pending = pltpu.async_copy(
                            y_hbm.at[src_v.at[j + 1]], gbuf.at[(j + 1) % 2],
                            gsem)
                    pltpu.sync_copy(gbuf.at[j % 2], acc.at[dst_v.at[j]],
                                    add=True)
            return 0

        with jax.named_scope("edge_loop"):
            lax.fori_loop(0, max(ng0, ng1), group, 0)

        with jax.named_scope("end_barrier"):
            plsc.subcore_barrier()

        with jax.named_scope("writeback"):
            wb = s * base
            pltpu.sync_copy(acc.at[pl.ds(wb, wlen)],
                            psum_hbm.at[c, pl.ds(wb, wlen)])

    return pl.kernel(body, out_type=out_type, mesh=mesh, scratch_types=scratch)


def _make_sc_counts(n_nodes, rows_per_tile):
    n_acc = n_nodes + PADR
    base = (n_nodes // NS) // 8 * 8
    wlen = n_nodes - base * (NS - 1)

    mesh = plsc.VectorSubcoreMesh(core_axis_name="c", subcore_axis_name="s")
    out_type = jax.ShapeDtypeStruct((NC, n_nodes, D), jnp.float32)
    scratch = [
        pltpu.VMEM((GRP, CH), jnp.int32),
        pltpu.VMEM((CH, D), jnp.float32),
        pltpu.VMEM_SHARED((n_acc, D), jnp.float32),
    ]

    def body(dst_hbm, cnt_hbm, dst_v, buf, cacc):
        c = lax.axis_index("c")
        s = lax.axis_index("s")
        row0 = (c * NS + s) * rows_per_tile

        def fill(val):
            def frow(i, _):
                for j in range(D // 16):
                    buf[i, pl.ds(j * 16, 16)] = jnp.full((16,), val, jnp.float32)
                return 0
            lax.fori_loop(0, CH, frow, 0)

        fill(0.0)
        zb = s * base
        for j in range(wlen // CH):
            pltpu.sync_copy(buf, cacc.at[pl.ds(zb + j * CH, CH)])
        @pl.when(s == 0)
        def _():
            pltpu.sync_copy(buf.at[pl.ds(0, PADR)], cacc.at[pl.ds(n_nodes, PADR)])
        fill(1.0)

        plsc.subcore_barrier()

        def group(g, _):
            pltpu.sync_copy(dst_hbm.at[pl.ds(row0 + g * GRP, GRP)], dst_v)
            for j in range(GRP):
                pltpu.sync_copy(buf, cacc.at[dst_v.at[j]], add=True)
            return 0

        lax.fori_loop(0, rows_per_tile // GRP, group, 0)

        plsc.subcore_barrier()
        wb = s * base
        pltpu.sync_copy(cacc.at[pl.ds(wb, wlen)],
                        cnt_hbm.at[c, pl.ds(wb, wlen)])

    return pl.kernel(body, out_type=out_type, mesh=mesh, scratch_types=scratch)


def kernel(x, edge_index, Wl0, bl0, Wr0, Wl1, bl1, Wr1, Wl2, bl2, Wr2):
    n = x.shape[0]
    e = edge_index.shape[1]

    e_pad = _round_up(e, NC * NS * CH * 8)
    src = edge_index[0]
    dst = edge_index[1]
    if e_pad != e:
        pad = e_pad - e
        src = jnp.concatenate(
            [src, jnp.arange(pad, dtype=jnp.int32) % n])
        dst = jnp.concatenate(
            [dst, n + (jnp.arange(pad, dtype=jnp.int32) % PADR)])
    src2d = src.reshape(e_pad // CH, CH)
    dst2d = dst.reshape(e_pad // CH, CH)
    rows_per_tile = (e_pad // CH) // (NC * NS)

    sc_seg = _make_sc_segsum(n, rows_per_tile)
    sc_cnt = _make_sc_counts(n, rows_per_tile)

    b0 = bl0.reshape(1, D)
    b1 = bl1.reshape(1, D)
    b2 = bl2.reshape(1, D)

    C = sc_cnt(dst2d)
    WL = jnp.stack([Wl0, Wl1, Wl2])
    WR = jnp.stack([Wr0, Wr1, Wr2])
    B3 = jnp.stack([b0, b1, b2])

    def step(h, wlb):
        Wl, Wr, b = wlb
        y, r = _tc_dense(h, Wl, Wr)
        P = sc_seg(y, src2d, dst2d)
        z, h2 = _tc_zcombine(P, C, r, b)
        return h2, z

    _, zs = lax.scan(step, x, (WL, WR, B3))
    return zs[2]


# speedup vs baseline: 2.7422x; 1.0461x over previous
import functools

import jax
import jax.numpy as jnp
from jax import lax
from jax.experimental import pallas as pl
from jax.experimental.pallas import tpu as pltpu
from jax.experimental.pallas import tpu_sc as plsc

N = 10000
D = 128
NC = 2
NS = 16
CH = 128
PADR = 128
GRP = 8


def _round_up(a, b):
    return (a + b - 1) // b * b


_ROWS = 2000


def _dense_body(h_ref, wl_ref, wr_ref, y_ref, r_ref):
    h = h_ref[...]
    y_ref[...] = jnp.dot(h, wl_ref[...], preferred_element_type=jnp.float32)
    r_ref[...] = jnp.dot(h, wr_ref[...], preferred_element_type=jnp.float32)


def _tc_dense(h, Wl, Wr):
    n = h.shape[0]
    return pl.pallas_call(
        _dense_body,
        grid=(n // _ROWS,),
        in_specs=[
            pl.BlockSpec((_ROWS, D), lambda i: (i, 0)),
            pl.BlockSpec((D, D), lambda i: (0, 0)),
            pl.BlockSpec((D, D), lambda i: (0, 0)),
        ],
        out_specs=[pl.BlockSpec((_ROWS, D), lambda i: (i, 0))] * 2,
        out_shape=[jax.ShapeDtypeStruct((n, D), jnp.float32)] * 2,
    )(h, Wl, Wr)


def _combine_body(p_ref, c_ref, rp_ref, b_ref, wl_ref, wr_ref, y_ref, r_ref):
    cnt = jnp.maximum(c_ref[0][:, 0:1] + c_ref[1][:, 0:1], 1.0)
    mean = (p_ref[0] + p_ref[1]) / cnt
    h = jnp.maximum(mean + rp_ref[...] + b_ref[...], 0.0)
    y_ref[...] = jnp.dot(h, wl_ref[...], preferred_element_type=jnp.float32)
    r_ref[...] = jnp.dot(h, wr_ref[...], preferred_element_type=jnp.float32)


def _tc_combine(P, C, r_prev, b, Wl, Wr):
    n = r_prev.shape[0]
    return pl.pallas_call(
        _combine_body,
        grid=(n // _ROWS,),
        in_specs=[
            pl.BlockSpec((2, _ROWS, D), lambda i: (0, i, 0)),
            pl.BlockSpec((2, _ROWS, D), lambda i: (0, i, 0)),
            pl.BlockSpec((_ROWS, D), lambda i: (i, 0)),
            pl.BlockSpec((1, D), lambda i: (0, 0)),
            pl.BlockSpec((D, D), lambda i: (0, 0)),
            pl.BlockSpec((D, D), lambda i: (0, 0)),
        ],
        out_specs=[pl.BlockSpec((_ROWS, D), lambda i: (i, 0))] * 2,
        out_shape=[jax.ShapeDtypeStruct((n, D), jnp.float32)] * 2,
    )(P, C, r_prev, b, Wl, Wr)


def _final_body(p_ref, c_ref, rp_ref, b_ref, o_ref):
    cnt = jnp.maximum(c_ref[0][:, 0:1] + c_ref[1][:, 0:1], 1.0)
    o_ref[...] = (p_ref[0] + p_ref[1]) / cnt + rp_ref[...] + b_ref[...]


def _tc_final(P, C, r_prev, b):
    n = r_prev.shape[0]
    return pl.pallas_call(
        _final_body,
        grid=(n // _ROWS,),
        in_specs=[
            pl.BlockSpec((2, _ROWS, D), lambda i: (0, i, 0)),
            pl.BlockSpec((2, _ROWS, D), lambda i: (0, i, 0)),
            pl.BlockSpec((_ROWS, D), lambda i: (i, 0)),
            pl.BlockSpec((1, D), lambda i: (0, 0)),
        ],
        out_specs=pl.BlockSpec((_ROWS, D), lambda i: (i, 0)),
        out_shape=jax.ShapeDtypeStruct((n, D), jnp.float32),
    )(P, C, r_prev, b)


def _zcomb_body(p_ref, c_ref, rp_ref, b_ref, z_ref, h_ref):
    cnt = jnp.maximum(c_ref[0][:, 0:1] + c_ref[1][:, 0:1], 1.0)
    z = (p_ref[0] + p_ref[1]) / cnt + rp_ref[...] + b_ref[...]
    z_ref[...] = z
    h_ref[...] = jnp.maximum(z, 0.0)


def _tc_zcombine(P, C, r_prev, b):
    n = r_prev.shape[0]
    return pl.pallas_call(
        _zcomb_body,
        grid=(n // _ROWS,),
        in_specs=[
            pl.BlockSpec((2, _ROWS, D), lambda i: (0, i, 0)),
            pl.BlockSpec((2, _ROWS, D), lambda i: (0, i, 0)),
            pl.BlockSpec((_ROWS, D), lambda i: (i, 0)),
            pl.BlockSpec((1, D), lambda i: (0, 0)),
        ],
        out_specs=[pl.BlockSpec((_ROWS, D), lambda i: (i, 0))] * 2,
        out_shape=[jax.ShapeDtypeStruct((n, D), jnp.float32)] * 2,
    )(P, C, r_prev, b)


def _make_sc_segsum(n_nodes, rows_per_tile, rpt0=None):
    n_acc = n_nodes + PADR
    base = (n_nodes // NS) // 8 * 8
    wlen = n_nodes - base * (NS - 1)
    assert wlen % CH == 0 and wlen % 8 == 0

    mesh = plsc.VectorSubcoreMesh(core_axis_name="c", subcore_axis_name="s")
    out_type = jax.ShapeDtypeStruct((NC, n_nodes, D), jnp.float32)

    scratch = [
        pltpu.VMEM((GRP, CH), jnp.int32),
        pltpu.VMEM((GRP, CH), jnp.int32),
        pltpu.VMEM((2, CH, D), jnp.float32),
        pltpu.VMEM_SHARED((n_acc, D), jnp.float32),
        pltpu.SemaphoreType.DMA,
    ]

    if rpt0 is None:
        rpt0 = rows_per_tile
    rpt1 = 2 * rows_per_tile - rpt0
    assert rpt0 % GRP == 0 and rpt1 % GRP == 0
    ng0, ng1 = rpt0 // GRP, rpt1 // GRP

    def body(y_hbm, src_hbm, dst_hbm, psum_hbm, src_v, dst_v, gbuf, acc, gsem):
        c = lax.axis_index("c")
        s = lax.axis_index("s")
        row0 = jnp.where(c == 0, s * rpt0, NS * rpt0 + s * rpt1)
        ng = jnp.where(c == 0, ng0, ng1)

        with jax.named_scope("zero_phase"):
            def zrow(i, _):
                for j in range(D // 16):
                    gbuf[0, i, pl.ds(j * 16, 16)] = jnp.zeros((16,),
                                                              jnp.float32)
                return 0

            lax.fori_loop(0, CH, zrow, 0)

            zb = s * base
            for j in range(wlen // CH):
                pltpu.sync_copy(gbuf.at[0], acc.at[pl.ds(zb + j * CH, CH)])
            @pl.when(s == 0)
            def _():
                pltpu.sync_copy(gbuf.at[0, pl.ds(0, PADR)],
                                acc.at[pl.ds(n_nodes, PADR)])

        with jax.named_scope("zero_barrier"):
            plsc.subcore_barrier()

        def group(g, _):
            @pl.when(g < ng)
            def _():
                gr = row0 + g * GRP
                pltpu.sync_copy(src_hbm.at[pl.ds(gr, GRP)], src_v)
                pltpu.sync_copy(dst_hbm.at[pl.ds(gr, GRP)], dst_v)
                pending = pltpu.async_copy(y_hbm.at[src_v.at[0]], gbuf.at[0],
                                           gsem)
                for j in range(GRP):
                    pending.wait()
                    if j < GRP - 1:
                        pending = pltpu.async_copy(
                            y_hbm.at[src_v.at[j + 1]], gbuf.at[(j + 1) % 2],
                            gsem)
                    pltpu.sync_copy(gbuf.at[j % 2], acc.at[dst_v.at[j]],
                                    add=True)
            return 0

        with jax.named_scope("edge_loop"):
            lax.fori_loop(0, max(ng0, ng1), group, 0)

        with jax.named_scope("end_barrier"):
            plsc.subcore_barrier()

        with jax.named_scope("writeback"):
            wb = s * base
            pltpu.sync_copy(acc.at[pl.ds(wb, wlen)],
                            psum_hbm.at[c, pl.ds(wb, wlen)])

    return pl.kernel(body, out_type=out_type, mesh=mesh, scratch_types=scratch)


def _make_sc_counts(n_nodes, rows_per_tile):
    n_acc = n_nodes + PADR
    base = (n_nodes // NS) // 8 * 8
    wlen = n_nodes - base * (NS - 1)

    mesh = plsc.VectorSubcoreMesh(core_axis_name="c", subcore_axis_name="s")
    out_type = jax.ShapeDtypeStruct((NC, n_nodes, D), jnp.float32)
    scratch = [
        pltpu.VMEM((GRP, CH), jnp.int32),
        pltpu.VMEM((CH, D), jnp.float32),
        pltpu.VMEM_SHARED((n_acc, D), jnp.float32),
    ]

    def body(dst_hbm, cnt_hbm, dst_v, buf, cacc):
        c = lax.axis_index("c")
        s = lax.axis_index("s")
        row0 = (c * NS + s) * rows_per_tile

        def fill(val):
            def frow(i, _):
                for j in range(D // 16):
                    buf[i, pl.ds(j * 16, 16)] = jnp.full((16,), val, jnp.float32)
                return 0
            lax.fori_loop(0, CH, frow, 0)

        fill(0.0)
        zb = s * base
        for j in range(wlen // CH):
            pltpu.sync_copy(buf, cacc.at[pl.ds(zb + j * CH, CH)])
        @pl.when(s == 0)
        def _():
            pltpu.sync_copy(buf.at[pl.ds(0, PADR)], cacc.at[pl.ds(n_nodes, PADR)])
        fill(1.0)

        plsc.subcore_barrier()

        def group(g, _):
            pltpu.sync_copy(dst_hbm.at[pl.ds(row0 + g * GRP, GRP)], dst_v)
            for j in range(GRP):
                pltpu.sync_copy(buf, cacc.at[dst_v.at[j]], add=True)
            return 0

        lax.fori_loop(0, rows_per_tile // GRP, group, 0)

        plsc.subcore_barrier()
        wb = s * base
        pltpu.sync_copy(cacc.at[pl.ds(wb, wlen)],
                        cnt_hbm.at[c, pl.ds(wb, wlen)])

    return pl.kernel(body, out_type=out_type, mesh=mesh, scratch_types=scratch)


def kernel(x, edge_index, Wl0, bl0, Wr0, Wl1, bl1, Wr1, Wl2, bl2, Wr2):
    n = x.shape[0]
    e = edge_index.shape[1]

    e_pad = _round_up(e, NC * NS * CH * 8)
    src = edge_index[0]
    dst = edge_index[1]
    if e_pad != e:
        pad = e_pad - e
        src = jnp.concatenate(
            [src, jnp.arange(pad, dtype=jnp.int32) % n])
        dst = jnp.concatenate(
            [dst, n + (jnp.arange(pad, dtype=jnp.int32) % PADR)])
    src2d = src.reshape(e_pad // CH, CH)
    dst2d = dst.reshape(e_pad // CH, CH)
    rows_per_tile = (e_pad // CH) // (NC * NS)

    sc_seg = _make_sc_segsum(n, rows_per_tile)
    sc_cnt = _make_sc_counts(n, rows_per_tile)

    b0 = bl0.reshape(1, D)
    b1 = bl1.reshape(1, D)
    b2 = bl2.reshape(1, D)

    C = sc_cnt(dst2d)
    y0, r0 = _tc_dense(x, Wl0, Wr0)
    P0 = sc_seg(y0, src2d, dst2d)
    y1, r1 = _tc_combine(P0, C, r0, b0, Wl1, Wr1)
    P1 = sc_seg(y1, src2d, dst2d)
    y2, r2 = _tc_combine(P1, C, r1, b1, Wl2, Wr2)
    P2 = sc_seg(y2, src2d, dst2d)
    return _tc_final(P2, C, r2, b2)


# speedup vs baseline: 2.8568x; 1.0418x over previous
import functools

import jax
import jax.numpy as jnp
from jax import lax
from jax.experimental import pallas as pl
from jax.experimental.pallas import tpu as pltpu
from jax.experimental.pallas import tpu_sc as plsc

N = 10000
D = 128
NC = 2
NS = 16
CH = 128
PADR = 128
GRP = 16


def _round_up(a, b):
    return (a + b - 1) // b * b


_ROWS = 2000


def _dense_body(h_ref, wl_ref, wr_ref, y_ref, r_ref):
    h = h_ref[...]
    y_ref[...] = jnp.dot(h, wl_ref[...], preferred_element_type=jnp.float32)
    r_ref[...] = jnp.dot(h, wr_ref[...], preferred_element_type=jnp.float32)


def _tc_dense(h, Wl, Wr):
    n = h.shape[0]
    return pl.pallas_call(
        _dense_body,
        grid=(n // _ROWS,),
        in_specs=[
            pl.BlockSpec((_ROWS, D), lambda i: (i, 0)),
            pl.BlockSpec((D, D), lambda i: (0, 0)),
            pl.BlockSpec((D, D), lambda i: (0, 0)),
        ],
        out_specs=[pl.BlockSpec((_ROWS, D), lambda i: (i, 0))] * 2,
        out_shape=[jax.ShapeDtypeStruct((n, D), jnp.float32)] * 2,
    )(h, Wl, Wr)


def _combine_body(p_ref, c_ref, rp_ref, b_ref, wl_ref, wr_ref, y_ref, r_ref):
    cnt = jnp.maximum(c_ref[0][:, 0:1] + c_ref[1][:, 0:1], 1.0)
    mean = (p_ref[0] + p_ref[1]) / cnt
    h = jnp.maximum(mean + rp_ref[...] + b_ref[...], 0.0)
    y_ref[...] = jnp.dot(h, wl_ref[...], preferred_element_type=jnp.float32)
    r_ref[...] = jnp.dot(h, wr_ref[...], preferred_element_type=jnp.float32)


def _tc_combine(P, C, r_prev, b, Wl, Wr):
    n = r_prev.shape[0]
    return pl.pallas_call(
        _combine_body,
        grid=(n // _ROWS,),
        in_specs=[
            pl.BlockSpec((2, _ROWS, D), lambda i: (0, i, 0)),
            pl.BlockSpec((2, _ROWS, D), lambda i: (0, i, 0)),
            pl.BlockSpec((_ROWS, D), lambda i: (i, 0)),
            pl.BlockSpec((1, D), lambda i: (0, 0)),
            pl.BlockSpec((D, D), lambda i: (0, 0)),
            pl.BlockSpec((D, D), lambda i: (0, 0)),
        ],
        out_specs=[pl.BlockSpec((_ROWS, D), lambda i: (i, 0))] * 2,
        out_shape=[jax.ShapeDtypeStruct((n, D), jnp.float32)] * 2,
    )(P, C, r_prev, b, Wl, Wr)


def _final_body(p_ref, c_ref, rp_ref, b_ref, o_ref):
    cnt = jnp.maximum(c_ref[0][:, 0:1] + c_ref[1][:, 0:1], 1.0)
    o_ref[...] = (p_ref[0] + p_ref[1]) / cnt + rp_ref[...] + b_ref[...]


def _tc_final(P, C, r_prev, b):
    n = r_prev.shape[0]
    return pl.pallas_call(
        _final_body,
        grid=(n // _ROWS,),
        in_specs=[
            pl.BlockSpec((2, _ROWS, D), lambda i: (0, i, 0)),
            pl.BlockSpec((2, _ROWS, D), lambda i: (0, i, 0)),
            pl.BlockSpec((_ROWS, D), lambda i: (i, 0)),
            pl.BlockSpec((1, D), lambda i: (0, 0)),
        ],
        out_specs=pl.BlockSpec((_ROWS, D), lambda i: (i, 0)),
        out_shape=jax.ShapeDtypeStruct((n, D), jnp.float32),
    )(P, C, r_prev, b)


def _zcomb_body(p_ref, c_ref, rp_ref, b_ref, z_ref, h_ref):
    cnt = jnp.maximum(c_ref[0][:, 0:1] + c_ref[1][:, 0:1], 1.0)
    z = (p_ref[0] + p_ref[1]) / cnt + rp_ref[...] + b_ref[...]
    z_ref[...] = z
    h_ref[...] = jnp.maximum(z, 0.0)


def _tc_zcombine(P, C, r_prev, b):
    n = r_prev.shape[0]
    return pl.pallas_call(
        _zcomb_body,
        grid=(n // _ROWS,),
        in_specs=[
            pl.BlockSpec((2, _ROWS, D), lambda i: (0, i, 0)),
            pl.BlockSpec((2, _ROWS, D), lambda i: (0, i, 0)),
            pl.BlockSpec((_ROWS, D), lambda i: (i, 0)),
            pl.BlockSpec((1, D), lambda i: (0, 0)),
        ],
        out_specs=[pl.BlockSpec((_ROWS, D), lambda i: (i, 0))] * 2,
        out_shape=[jax.ShapeDtypeStruct((n, D), jnp.float32)] * 2,
    )(P, C, r_prev, b)


def _make_sc_segsum(n_nodes, rows_per_tile, rpt0=None):
    n_acc = n_nodes + PADR
    base = (n_nodes // NS) // 8 * 8
    wlen = n_nodes - base * (NS - 1)
    assert wlen % CH == 0 and wlen % 8 == 0

    mesh = plsc.VectorSubcoreMesh(core_axis_name="c", subcore_axis_name="s")
    out_type = jax.ShapeDtypeStruct((NC, n_nodes, D), jnp.float32)

    scratch = [
        pltpu.VMEM((GRP, CH), jnp.int32),
        pltpu.VMEM((GRP, CH), jnp.int32),
        pltpu.VMEM((2, CH, D), jnp.float32),
        pltpu.VMEM_SHARED((n_acc, D), jnp.float32),
        pltpu.SemaphoreType.DMA,
    ]

    if rpt0 is None:
        rpt0 = rows_per_tile
    rpt1 = 2 * rows_per_tile - rpt0
    assert rpt0 % GRP == 0 and rpt1 % GRP == 0
    ng0, ng1 = rpt0 // GRP, rpt1 // GRP

    def body(y_hbm, src_hbm, dst_hbm, psum_hbm, src_v, dst_v, gbuf, acc, gsem):
        c = lax.axis_index("c")
        s = lax.axis_index("s")
        row0 = jnp.where(c == 0, s * rpt0, NS * rpt0 + s * rpt1)
        ng = jnp.where(c == 0, ng0, ng1)

        with jax.named_scope("zero_phase"):
            def zrow(i, _):
                for j in range(D // 16):
                    gbuf[0, i, pl.ds(j * 16, 16)] = jnp.zeros((16,),
                                                              jnp.float32)
                return 0

            lax.fori_loop(0, CH, zrow, 0)

            zb = s * base
            for j in range(wlen // CH):
                pltpu.sync_copy(gbuf.at[0], acc.at[pl.ds(zb + j * CH, CH)])
            @pl.when(s == 0)
            def _():
                pltpu.sync_copy(gbuf.at[0, pl.ds(0, PADR)],
                                acc.at[pl.ds(n_nodes, PADR)])

        with jax.named_scope("zero_barrier"):
            plsc.subcore_barrier()

        def group(g, _):
            @pl.when(g < ng)
            def _():
                gr = row0 + g * GRP
                pltpu.sync_copy(src_hbm.at[pl.ds(gr, GRP)], src_v)
                pltpu.sync_copy(dst_hbm.at[pl.ds(gr, GRP)], dst_v)
                pending = pltpu.async_copy(y_hbm.at[src_v.at[0]], gbuf.at[0],
                                           gsem)
                for j in range(GRP):
                    pending.wait()
                    if j < GRP - 1:
                        pending = pltpu.async_copy(
                            y_hbm.at[src_v.at[j + 1]], gbuf.at[(j + 1) % 2],
                            gsem)
                    pltpu.sync_copy(gbuf.at[j % 2], acc.at[dst_v.at[j]],
                                    add=True)
            return 0

        with jax.named_scope("edge_loop"):
            lax.fori_loop(0, max(ng0, ng1), group, 0)

        with jax.named_scope("end_barrier"):
            plsc.subcore_barrier()

        with jax.named_scope("writeback"):
            wb = s * base
            pltpu.sync_copy(acc.at[pl.ds(wb, wlen)],
                            psum_hbm.at[c, pl.ds(wb, wlen)])

    return pl.kernel(body, out_type=out_type, mesh=mesh, scratch_types=scratch)


def _make_sc_counts(n_nodes, rows_per_tile):
    n_acc = n_nodes + PADR
    base = (n_nodes // NS) // 8 * 8
    wlen = n_nodes - base * (NS - 1)

    mesh = plsc.VectorSubcoreMesh(core_axis_name="c", subcore_axis_name="s")
    out_type = jax.ShapeDtypeStruct((NC, n_nodes, D), jnp.float32)
    scratch = [
        pltpu.VMEM((GRP, CH), jnp.int32),
        pltpu.VMEM((CH, D), jnp.float32),
        pltpu.VMEM_SHARED((n_acc, D), jnp.float32),
    ]

    def body(dst_hbm, cnt_hbm, dst_v, buf, cacc):
        c = lax.axis_index("c")
        s = lax.axis_index("s")
        row0 = (c * NS + s) * rows_per_tile

        def fill(val):
            def frow(i, _):
                for j in range(D // 16):
                    buf[i, pl.ds(j * 16, 16)] = jnp.full((16,), val, jnp.float32)
                return 0
            lax.fori_loop(0, CH, frow, 0)

        fill(0.0)
        zb = s * base
        for j in range(wlen // CH):
            pltpu.sync_copy(buf, cacc.at[pl.ds(zb + j * CH, CH)])
        @pl.when(s == 0)
        def _():
            pltpu.sync_copy(buf.at[pl.ds(0, PADR)], cacc.at[pl.ds(n_nodes, PADR)])
        fill(1.0)

        plsc.subcore_barrier()

        def group(g, _):
            pltpu.sync_copy(dst_hbm.at[pl.ds(row0 + g * GRP, GRP)], dst_v)
            for j in range(GRP):
                pltpu.sync_copy(buf, cacc.at[dst_v.at[j]], add=True)
            return 0

        lax.fori_loop(0, rows_per_tile // GRP, group, 0)

        plsc.subcore_barrier()
        wb = s * base
        pltpu.sync_copy(cacc.at[pl.ds(wb, wlen)],
                        cnt_hbm.at[c, pl.ds(wb, wlen)])

    return pl.kernel(body, out_type=out_type, mesh=mesh, scratch_types=scratch)


def kernel(x, edge_index, Wl0, bl0, Wr0, Wl1, bl1, Wr1, Wl2, bl2, Wr2):
    n = x.shape[0]
    e = edge_index.shape[1]

    e_pad = _round_up(e, NC * NS * CH * 8)
    src = edge_index[0]
    dst = edge_index[1]
    if e_pad != e:
        pad = e_pad - e
        src = jnp.concatenate(
            [src, jnp.arange(pad, dtype=jnp.int32) % n])
        dst = jnp.concatenate(
            [dst, n + (jnp.arange(pad, dtype=jnp.int32) % PADR)])
    src2d = src.reshape(e_pad // CH, CH)
    dst2d = dst.reshape(e_pad // CH, CH)
    rows_per_tile = (e_pad // CH) // (NC * NS)

    sc_seg = _make_sc_segsum(n, rows_per_tile)
    sc_cnt = _make_sc_counts(n, rows_per_tile)

    b0 = bl0.reshape(1, D)
    b1 = bl1.reshape(1, D)
    b2 = bl2.reshape(1, D)

    C = sc_cnt(dst2d)
    y0, r0 = _tc_dense(x, Wl0, Wr0)
    P0 = sc_seg(y0, src2d, dst2d)
    y1, r1 = _tc_combine(P0, C, r0, b0, Wl1, Wr1)
    P1 = sc_seg(y1, src2d, dst2d)
    y2, r2 = _tc_combine(P1, C, r1, b1, Wl2, Wr2)
    P2 = sc_seg(y2, src2d, dst2d)
    return _tc_final(P2, C, r2, b2)
